# baseline (speedup 1.0000x reference)
import functools

import jax
import jax.numpy as jnp
from jax import lax
from jax.experimental import pallas as pl
from jax.experimental.pallas import tpu as pltpu
from jax.experimental.pallas import tpu_sc as plsc

B, S, D, V = 1024, 200, 128, 100000
N = B * S
NC, NS = 2, 16
NW = NC * NS
RPW = N // NW
C = 64
NCHUNK = RPW // C
NBUF = 4


def _combine_body(pe_ref, seg_ref, out_ref):
    peq = pe_ref[...] + seg_ref[...][0][None, :]
    dvec = (seg_ref[...][1] - seg_ref[...][0])[None, :]
    out_ref[...] = jnp.concatenate(
        [peq, jnp.broadcast_to(dvec, (8, D))], axis=0)


def _make_tables(pe, segment_table):
    return pl.pallas_call(
        _combine_body,
        out_shape=jax.ShapeDtypeStruct((S + 8, D), jnp.float32),
    )(pe, segment_table)


def _sc_body(x_hbm, seg_hbm, tok_tab_hbm, peq_hbm, out_hbm,
             xall, segall, pe_res, d8,
             xidx0, xidx1, xidx2, xidx3,
             tok0, tok1, tok2, tok3,
             gsem0, gsem1, gsem2, gsem3,
             wsem0, wsem1, wsem2, wsem3):
    wid = lax.axis_index("s") * NC + lax.axis_index("c")
    base = wid * RPW
    xidx = (xidx0, xidx1, xidx2, xidx3)
    tok = (tok0, tok1, tok2, tok3)
    gsem = (gsem0, gsem1, gsem2, gsem3)
    wsem = (wsem0, wsem1, wsem2, wsem3)

    pltpu.sync_copy(x_hbm.at[pl.ds(base, RPW)], xall)
    pltpu.sync_copy(seg_hbm.at[pl.ds(base, RPW)], segall.at[pl.ds(0, RPW)])
    pltpu.sync_copy(peq_hbm.at[pl.ds(0, S)], pe_res)
    pltpu.sync_copy(peq_hbm.at[pl.ds(S, 8)], d8)

    def prep(chunk, b):
        lbase = chunk * C
        for j in range(C // 16):
            xidx[b][pl.ds(j * 16, 16)] = xall[pl.ds(lbase + j * 16, 16)]
        pltpu.async_copy(tok_tab_hbm.at[xidx[b]], tok[b], gsem[b])

    def wait_g(b):
        pltpu.make_async_copy(tok_tab_hbm.at[xidx[b]], tok[b], gsem[b]).wait()

    def add_rows(chunk, b):
        t = tok[b]
        lbase = chunk * C
        pos0 = lax.rem(base + lbase, S)
        dv = [d8[0, pl.ds(j * 16, 16)] for j in range(D // 16)]

        def row_body(i, acc):
            for u in range(4):
                r = i * 4 + u
                sv = segall[pl.ds(lbase + r, 16)]
                segf = jnp.full((16,), sv[0], jnp.float32)
                p = pos0 + r
                off = lax.select(p >= S, p - S, p)
                for j in range(D // 16):
                    sl = pl.ds(j * 16, 16)
                    plsc.addupdate(t.at[r, sl],
                                   pe_res[off, sl] + segf * dv[j])
            return acc

        lax.fori_loop(0, C // 4, row_body, 0)

    def fire_w(chunk, b):
        gbase = base + chunk * C
        pltpu.async_copy(tok[b], out_hbm.at[pl.ds(gbase, C)], wsem[b])

    def wait_w(b):
        pltpu.make_async_copy(tok[b], out_hbm.at[pl.ds(0, C)], wsem[b]).wait()

    for b in range(NBUF):
        prep(b, b)

    def body(k, acc):
        c0 = NBUF * k
        for b in range(NBUF):
            wait_g(b)
            add_rows(c0 + b, b)
            fire_w(c0 + b, b)
        for b in range(NBUF):
            wait_w(b)
            prep(c0 + NBUF + b, b)
        return acc

    lax.fori_loop(0, NCHUNK // NBUF - 1, body, 0)
    for b in range(NBUF):
        wait_g(b)
        add_rows(NCHUNK - NBUF + b, b)
        fire_w(NCHUNK - NBUF + b, b)
    for b in range(NBUF):
        wait_w(b)


def kernel(x, segment_tokens, token_table, segment_table, pe):
    tables = _make_tables(pe.astype(jnp.float32),
                          segment_table.astype(jnp.float32))
    x_flat = x.reshape(N).astype(jnp.int32)
    seg_flat = segment_tokens.reshape(N).astype(jnp.int32)

    mesh = plsc.VectorSubcoreMesh(core_axis_name="c", subcore_axis_name="s")
    sc = functools.partial(
        pl.kernel,
        mesh=mesh,
        out_type=jax.ShapeDtypeStruct((N, D), jnp.float32),
        scratch_types=(
            [pltpu.VMEM((RPW,), jnp.int32)]
            + [pltpu.VMEM((RPW + 16,), jnp.int32)]
            + [pltpu.VMEM((S, D), jnp.float32)]
            + [pltpu.VMEM((8, D), jnp.float32)]
            + [pltpu.VMEM((C,), jnp.int32)] * 4
            + [pltpu.VMEM((C, D), jnp.float32)] * 4
            + [pltpu.SemaphoreType.DMA] * 8
        ),
    )(_sc_body)
    out = sc(x_flat, seg_flat, token_table, tables)
    return out.reshape(B, S, D)


# speedup vs baseline: 1.8801x; 1.8801x over previous
import functools

import jax
import jax.numpy as jnp
from jax import lax
from jax.experimental import pallas as pl
from jax.experimental.pallas import tpu as pltpu
from jax.experimental.pallas import tpu_sc as plsc

B, S, D, V = 1024, 200, 128, 100000
N = B * S
NC, NS = 2, 16
NW = NC * NS
RPW = N // NW
C = 64
NCHUNK = RPW // C
NBUF = 4


def _combine_body(pe_ref, seg_ref, out_ref):
    out_ref[...] = seg_ref[...][:, None, :] + pe_ref[...][None, :, :]


def _make_combined(pe, segment_table):
    out = pl.pallas_call(
        _combine_body,
        out_shape=jax.ShapeDtypeStruct((2, S, D), jnp.float32),
    )(pe, segment_table)
    return out.reshape(2 * S, D)


def _sc_body(x_hbm, seg_hbm, tok_tab_hbm, comb_hbm, out_hbm,
             xall, segall,
             xidx0, xidx1, xidx2, xidx3,
             idxa, idxb, idxc, idxd,
             tok0, tok1, tok2, tok3,
             add0, add1, add2, add3,
             gsem0, gsem1, gsem2, gsem3,
             wsem0, wsem1, wsem2, wsem3):
    wid = lax.axis_index("s") * NC + lax.axis_index("c")
    base = wid * RPW
    xidx = (xidx0, xidx1, xidx2, xidx3)
    idx2 = (idxa, idxb, idxc, idxd)
    tok = (tok0, tok1, tok2, tok3)
    add = (add0, add1, add2, add3)
    gsem = (gsem0, gsem1, gsem2, gsem3)
    wsem = (wsem0, wsem1, wsem2, wsem3)

    pltpu.sync_copy(x_hbm.at[pl.ds(base, RPW)], xall)
    pltpu.sync_copy(seg_hbm.at[pl.ds(base, RPW)], segall)

    def prep(chunk, b):
        lbase = chunk * C
        for j in range(C // 16):
            xidx[b][pl.ds(j * 16, 16)] = xall[pl.ds(lbase + j * 16, 16)]
        pltpu.async_copy(tok_tab_hbm.at[xidx[b]], tok[b], gsem[b])
        for j in range(C // 16):
            rowid = base + lbase + j * 16 + lax.iota(jnp.int32, 16)
            pos = lax.rem(rowid, S)
            idx2[b][pl.ds(j * 16, 16)] = (
                segall[pl.ds(lbase + j * 16, 16)] * S + pos)
        pltpu.async_copy(comb_hbm.at[idx2[b]], add[b], gsem[b])

    def wait_g(b):
        pltpu.make_async_copy(tok_tab_hbm.at[xidx[b]], tok[b], gsem[b]).wait()
        pltpu.make_async_copy(comb_hbm.at[idx2[b]], add[b], gsem[b]).wait()

    def add_rows(b):
        t, a = tok[b], add[b]

        def row_body(i, acc):
            for u in range(4):
                r = i * 4 + u
                for j in range(D // 16):
                    sl = pl.ds(j * 16, 16)
                    plsc.addupdate(t.at[r, sl], a[r, sl])
            return acc

        lax.fori_loop(0, C // 4, row_body, 0)

    def fire_w(chunk, b):
        gbase = base + chunk * C
        pltpu.async_copy(tok[b], out_hbm.at[pl.ds(gbase, C)], wsem[b])

    def wait_w(b):
        pltpu.make_async_copy(tok[b], out_hbm.at[pl.ds(0, C)], wsem[b]).wait()

    for b in range(NBUF):
        prep(b, b)

    def body(k, acc):
        c0 = NBUF * k
        for b in range(NBUF):
            wait_g(b)
            add_rows(b)
            fire_w(c0 + b, b)
        for b in range(NBUF):
            wait_w(b)
            prep(c0 + NBUF + b, b)
        return acc

    lax.fori_loop(0, NCHUNK // NBUF - 1, body, 0)
    for b in range(NBUF):
        wait_g(b)
        add_rows(b)
        fire_w(NCHUNK - NBUF + b, b)
    for b in range(NBUF):
        wait_w(b)


def kernel(x, segment_tokens, token_table, segment_table, pe):
    combined = _make_combined(pe.astype(jnp.float32),
                              segment_table.astype(jnp.float32))
    x_flat = x.reshape(N).astype(jnp.int32)
    seg_flat = segment_tokens.reshape(N).astype(jnp.int32)

    mesh = plsc.VectorSubcoreMesh(core_axis_name="c", subcore_axis_name="s")
    sc = functools.partial(
        pl.kernel,
        mesh=mesh,
        out_type=jax.ShapeDtypeStruct((N, D), jnp.float32),
        scratch_types=(
            [pltpu.VMEM((RPW,), jnp.int32)] * 2
            + [pltpu.VMEM((C,), jnp.int32)] * 8
            + [pltpu.VMEM((C, D), jnp.float32)] * 8
            + [pltpu.SemaphoreType.DMA] * 8
        ),
    )(_sc_body)
    out = sc(x_flat, seg_flat, token_table, combined)
    return out.reshape(B, S, D)
